# Initial kernel scaffold; baseline (speedup 1.0000x reference)
import jax
import jax.numpy as jnp
from jax.experimental import pallas as pl


def kernel(subimg, base_image):
    raise NotImplementedError("write your pallas kernel here")


# speedup vs baseline: 2748.8987x; 2748.8987x over previous
import jax
import jax.numpy as jnp
from jax.experimental import pallas as pl

_TILE = 128
_SPATIAL = 512
_NTILES = _SPATIAL // _TILE
_SHIFT_TILES = 128 // _TILE


def _body(sub_ref, base_ref, out_ref):
    out_ref[...] = sub_ref[...] + base_ref[...]


def kernel(subimg, base_image):
    batch, chans, h, w = subimg.shape
    grid = (_NTILES, _NTILES)

    sub_spec = pl.BlockSpec(
        (batch, chans, _TILE, _TILE),
        lambda i, j: (0, 0, (i - _SHIFT_TILES) % _NTILES,
                      (j - _SHIFT_TILES) % _NTILES),
    )
    base_spec = pl.BlockSpec(
        (1, chans, _TILE, _TILE),
        lambda i, j: (0, 0, i, j),
    )
    out_spec = pl.BlockSpec(
        (batch, chans, _TILE, _TILE),
        lambda i, j: (0, 0, i, j),
    )

    return pl.pallas_call(
        _body,
        grid=grid,
        in_specs=[sub_spec, base_spec],
        out_specs=out_spec,
        out_shape=jax.ShapeDtypeStruct((batch, chans, h, w), subimg.dtype),
    )(subimg, base_image)
